# baseline (speedup 1.0000x reference)
import jax
import jax.numpy as jnp
import numpy as np
from jax import lax
from jax.experimental import pallas as pl
from jax.experimental.pallas import tpu as pltpu
from jax.experimental.pallas import tpu_sc as plsc

N_A = 10000
N_C2 = 2000
HID = 128
HEADS = 4
E_B = 320000
E_I2 = 32000

NC = 2
NS = 16

_CP = pltpu.CompilerParams(use_tc_tiling_on_sc=False, needs_layout_passes=False)


def _mesh():
    return plsc.VectorSubcoreMesh(core_axis_name="c", subcore_axis_name="s",
                                  num_cores=NC, num_subcores=NS)


def _zero16(ref, n):
    cgrp = ref.shape[1] // 16

    def it(i, _):
        ref[i // cgrp, pl.ds((i % cgrp) * 16, 16)] = jnp.zeros((16,), jnp.float32)
        return 0
    lax.fori_loop(0, n * cgrp, it, 0)


def _make_sc_rowsum(N, C, E, B, name):
    assert C % 16 == 0 and (C * 4) % 64 == 0 and N % 16 == 0
    assert E % (NC * NS * B) == 0 and B <= 128 and B % 8 == 0
    nchunks = N // 16
    iters = (nchunks + NS - 1) // NS
    e_per_tile = E // (NC * NS)
    nb = e_per_tile // B

    def body(table, src, dst, out, acc, zv, sv, dv, rows, sem):
        c = lax.axis_index("c")
        s = lax.axis_index("s")
        _zero16(zv, 16)

        def zchunk(i, _):
            k = i * NS + s

            @pl.when(k < nchunks)
            def _():
                pltpu.sync_copy(zv, acc.at[pl.ds(k * 16, 16)])
            return 0
        lax.fori_loop(0, iters, zchunk, 0)
        plsc.subcore_barrier()

        def batch(j, _):
            e0 = c * (E // NC) + s * e_per_tile + j * B
            pltpu.sync_copy(src.at[pl.ds(e0, B)], sv)
            pltpu.sync_copy(dst.at[pl.ds(e0, B)], dv)
            pltpu.async_copy(table.at[sv], rows, sem).wait()
            pltpu.sync_copy(rows, acc.at[dv], add=True)
            return 0
        lax.fori_loop(0, nb, batch, 0)
        plsc.subcore_barrier()

        def wchunk(i, _):
            k = i * NS + s

            @pl.when(k < nchunks)
            def _():
                pltpu.sync_copy(acc.at[pl.ds(k * 16, 16)],
                                out.at[c, pl.ds(k * 16, 16)])
            return 0
        lax.fori_loop(0, iters, wchunk, 0)

    return pl.kernel(
        body,
        out_type=jax.ShapeDtypeStruct((NC, N, C), jnp.float32),
        mesh=_mesh(),
        compiler_params=_CP,
        scratch_types=[
            pltpu.VMEM_SHARED((N, C), jnp.float32),
            pltpu.VMEM((16, C), jnp.float32),
            pltpu.VMEM((B,), jnp.int32),
            pltpu.VMEM((B,), jnp.int32),
            pltpu.VMEM((B, C), jnp.float32),
            pltpu.SemaphoreType.DMA,
        ],
        name=name,
    )


def _make_sc_gat(E, B, name):
    assert E % (NS * B) == 0 and B % 16 == 0
    e_per_tile = E // NS
    nb = e_per_tile // B
    nchunks = N_A // 16
    iters = (nchunks + NS - 1) // NS

    def body(X, eler, srcc, dstc, U, den,
             accU, accD, zv, sv, dv, elsrc, eldst, exb, rows, smX, smS, smD):
        c = lax.axis_index("c")
        s = lax.axis_index("s")
        iot = lax.iota(jnp.int32, 16)
        _zero16(zv, 16)

        def zex(i, _):
            exb[i, :] = jnp.zeros((16,), jnp.float32)
            return 0
        lax.fori_loop(0, B, zex, 0)

        def zchunk(i, _):
            k = i * NS + s

            @pl.when(k < nchunks)
            def _():
                pltpu.sync_copy(zv, accU.at[pl.ds(k * 16, 16)])
                pltpu.sync_copy(zv.at[:, pl.ds(0, 16)],
                                accD.at[pl.ds(k * 16, 16)])
            return 0
        lax.fori_loop(0, iters, zchunk, 0)
        plsc.subcore_barrier()

        def batch(j, _):
            e0 = c * E + s * e_per_tile + j * B
            pltpu.sync_copy(srcc.at[pl.ds(e0, B)], sv)
            pltpu.sync_copy(dstc.at[pl.ds(e0, B)], dv)
            cpX = pltpu.async_copy(X.at[c].at[sv], rows, smX)
            cpS = pltpu.async_copy(eler.at[c].at[sv], elsrc, smS)
            cpD = pltpu.async_copy(eler.at[c].at[dv], eldst, smD)
            cpX.wait()
            cpS.wait()
            cpD.wait()
            for i5 in range(B // 16):
                ridx = i5 * 16 + iot
                for h in range(4):
                    el_s = plsc.load_gather(
                        elsrc, [ridx, jnp.full((16,), h, jnp.int32)])
                    er_d = plsc.load_gather(
                        eldst, [ridx, jnp.full((16,), h + 4, jnp.int32)])
                    z = el_s + er_d
                    z = jnp.where(z >= 0.0, z, 0.2 * z)
                    plsc.store_scatter(
                        exb, [ridx, jnp.full((16,), h, jnp.int32)], jnp.exp(z))
            pltpu.sync_copy(exb, accD.at[dv], add=True)

            def scale(i, _):
                for h in range(4):
                    sc = plsc.load_gather(exb, [jnp.full((16,), i, jnp.int32),
                                                jnp.full((16,), h, jnp.int32)])
                    for gg in (2 * h, 2 * h + 1):
                        v = rows[i, pl.ds(gg * 16, 16)]
                        rows[i, pl.ds(gg * 16, 16)] = v * sc
                return 0
            lax.fori_loop(0, B, scale, 0)
            pltpu.sync_copy(rows, accU.at[dv], add=True)
            return 0
        lax.fori_loop(0, nb, batch, 0)
        plsc.subcore_barrier()

        def wchunk(i, _):
            k = i * NS + s

            @pl.when(k < nchunks)
            def _():
                pltpu.sync_copy(accU.at[pl.ds(k * 16, 16)],
                                U.at[c, pl.ds(k * 16, 16)])
                pltpu.sync_copy(accD.at[pl.ds(k * 16, 16)],
                                den.at[c, pl.ds(k * 16, 16)])
            return 0
        lax.fori_loop(0, iters, wchunk, 0)

    return pl.kernel(
        body,
        out_type=(jax.ShapeDtypeStruct((NC, N_A, 128), jnp.float32),
                  jax.ShapeDtypeStruct((NC, N_A, 16), jnp.float32)),
        mesh=_mesh(),
        compiler_params=_CP,
        scratch_types=[
            pltpu.VMEM_SHARED((N_A, 128), jnp.float32),
            pltpu.VMEM_SHARED((N_A, 16), jnp.float32),
            pltpu.VMEM((16, 128), jnp.float32),
            pltpu.VMEM((B,), jnp.int32),
            pltpu.VMEM((B,), jnp.int32),
            pltpu.VMEM((B, 16), jnp.float32),
            pltpu.VMEM((B, 16), jnp.float32),
            pltpu.VMEM((B, 16), jnp.float32),
            pltpu.VMEM((B, 128), jnp.float32),
            pltpu.SemaphoreType.DMA,
            pltpu.SemaphoreType.DMA,
            pltpu.SemaphoreType.DMA,
        ],
        name=name,
    )


def _make_sc_segmax(name):
    def body(h, dstg, out, colsv, dvv, acc):
        c = lax.axis_index("c")
        s = lax.axis_index("s")
        iot = lax.iota(jnp.int32, 16)
        msk = iot < 8
        pltpu.sync_copy(h.at[c, :, pl.ds(8 * s, 8)], colsv)
        pltpu.sync_copy(dstg.at[pl.ds(0, N_A)], dvv)

        def z(i, _):
            acc[i, :] = jnp.zeros((16,), jnp.float32)
            return 0
        lax.fori_loop(0, N_C2, z, 0)

        def it(i, _):
            ivec = jnp.full((16,), i, jnp.int32)
            dvec = plsc.load_gather(dvv, [ivec])
            dval = plsc.load_gather(colsv, [ivec, iot], mask=msk)
            av = plsc.load_gather(acc, [dvec, iot], mask=msk)
            plsc.store_scatter(acc, [dvec, iot], jnp.maximum(av, dval), mask=msk)
            return 0
        lax.fori_loop(0, N_A, it, 0)
        pltpu.sync_copy(acc.at[:, pl.ds(0, 8)], out.at[c, :, pl.ds(8 * s, 8)])

    return pl.kernel(
        body,
        out_type=jax.ShapeDtypeStruct((NC, N_C2, 128), jnp.float32),
        mesh=_mesh(),
        compiler_params=_CP,
        scratch_types=[
            pltpu.VMEM((N_A, 8), jnp.float32),
            pltpu.VMEM((N_A,), jnp.int32),
            pltpu.VMEM((N_C2, 16), jnp.float32),
        ],
        name=name,
    )


_sc_rowsum_A = _make_sc_rowsum(N_A, 128, E_B, 80, "sc_rowsum_a")
_sc_rowsum_C272 = _make_sc_rowsum(N_C2, 272, E_I2, 40, "sc_rowsum_c272")
_sc_rowsum_C128 = _make_sc_rowsum(N_C2, 128, E_I2, 40, "sc_rowsum_c128")
_sc_gat = _make_sc_gat(E_B, 80, "sc_gat")
_sc_segmax = _make_sc_segmax("sc_segmax")


def _mm(a, b):
    return jnp.matmul(a, b, precision='highest')


def _xe_body(f_ref, W_ref, A_ref, x_o, e_o):
    Xt = _mm(f_ref[:], W_ref[:])
    x_o[:] = Xt
    e_o[:] = _mm(Xt, A_ref[:])


def _hgat_body(u_ref, den_ref, b_ref, M_ref, o_ref):
    rec = 1.0 / jnp.maximum(den_ref[:], 1e-9)
    recx = _mm(rec, M_ref[:])
    o_ref[:] = jnp.maximum(u_ref[:] * recx + b_ref[:], 0.0)


def _gin_body(xg_ref, xf_ref, gp_ref, fp_ref, W1a, W1b, b1, g1, be1, W2, b2,
              h_o, s_o):
    xg = xg_ref[:] + gp_ref[0] + gp_ref[1]
    xf = xf_ref[:] + fp_ref[0] + fp_ref[1]
    h = _mm(xg, W1a[:]) + _mm(xf, W1b[:]) + b1[:]
    mu = jnp.mean(h, axis=0)
    d = h - mu
    var = jnp.mean(d * d, axis=0)
    a = jnp.maximum(d / jnp.sqrt(var + 1e-5) * g1[:] + be1[:], 0.0)
    o = jnp.maximum(_mm(a, W2[:]) + b2[:], 0.0)
    h_o[:] = o
    s_o[:] = jnp.sum(o, axis=0, keepdims=True)


def _ginc_body(x_ref, ap_ref, W1, b1, g1, be1, W2, b2, h_o, s_o):
    x = (x_ref[:] + ap_ref[0] + ap_ref[1])[:, :260]
    h = _mm(x, W1[:]) + b1[:]
    mu = jnp.mean(h, axis=0)
    d = h - mu
    var = jnp.mean(d * d, axis=0)
    a = jnp.maximum(d / jnp.sqrt(var + 1e-5) * g1[:] + be1[:], 0.0)
    o = jnp.maximum(_mm(a, W2[:]) + b2[:], 0.0)
    h_o[:] = o
    s_o[:] = jnp.sum(o, axis=0, keepdims=True)


def _out_mlp_kernel(hh_ref, w1_ref, b1_ref, w2_ref, b2_ref, o_ref):
    h = jnp.maximum(_mm(hh_ref[:], w1_ref[:]) + b1_ref[:], 0.0)
    o_ref[:] = _mm(h, w2_ref[:]) + b2_ref[:]


def _tc(body, outs, *args):
    return pl.pallas_call(body, out_shape=outs)(*args)


def kernel(feats_A, pca_C2, rni, params, edge_B1, edge_B2, dst_G1, edge_I2):
    p = params
    f32 = jnp.float32

    feats = jnp.concatenate([feats_A, rni], axis=-1)
    srccat = jnp.concatenate([edge_B1[0], edge_B2[0]])
    dstcat = jnp.concatenate([edge_B1[1], edge_B2[1]])

    Mnp = np.zeros((16, 128), np.float32)
    for h in range(4):
        Mnp[h, 32 * h:32 * h + 32] = 1.0
    M = jnp.asarray(Mnp)

    Xs, elers = [], []
    for t in range(2):
        al, ar = p['gat%d_al' % t], p['gat%d_ar' % t]
        A = jnp.zeros((128, 128), f32)
        for h in range(4):
            A = lax.dynamic_update_slice(A, al[h][:, None], (32 * h, h))
            A = lax.dynamic_update_slice(A, ar[h][:, None], (32 * h, h + 4))
        Xt, et = _tc(_xe_body,
                     (jax.ShapeDtypeStruct((N_A, 128), f32),
                      jax.ShapeDtypeStruct((N_A, 128), f32)),
                     feats, p['gat%d_W' % t], A)
        Xs.append(Xt)
        elers.append(et[:, :16])
    X = jnp.stack(Xs)
    eler = jnp.stack(elers)

    U, den = _sc_gat(X, eler, srccat, dstcat)

    hs, hsums = [], []
    for t, ei in enumerate([edge_B1, edge_B2]):
        src, dst = ei[0], ei[1]
        hgat_t = _tc(_hgat_body, jax.ShapeDtypeStruct((N_A, 128), f32),
                     U[t], den[t], p['gat%d_b' % t][None, :], M)
        gp = _sc_rowsum_A(hgat_t, src, dst)
        fp = _sc_rowsum_A(feats, src, dst)
        W1 = p['gin%d_W1' % t]
        h_t, hsum_t = _tc(
            _gin_body,
            (jax.ShapeDtypeStruct((N_A, 128), f32),
             jax.ShapeDtypeStruct((1, 128), f32)),
            hgat_t, feats, gp, fp, W1[:128], W1[128:],
            p['gin%d_b1' % t][None, :], p['gin%d_g1' % t][None, :],
            p['gin%d_be1' % t][None, :], p['gin%d_W2' % t],
            p['gin%d_b2' % t][None, :])
        hs.append(h_t)
        hsums.append(hsum_t)

    hstack = jnp.stack(hs)
    hC = _sc_segmax(hstack, dst_G1)

    h272 = jnp.concatenate(
        [hC[0], hC[1], pca_C2[..., :4], jnp.zeros((N_C2, 12), f32)], axis=1)
    ap = _sc_rowsum_C272(h272, edge_I2[0], edge_I2[1])
    g, _ = _tc(_ginc_body,
               (jax.ShapeDtypeStruct((N_C2, 128), f32),
                jax.ShapeDtypeStruct((1, 128), f32)),
               h272, ap, p['h2c0_W1'], p['h2c0_b1'][None, :],
               p['h2c0_g1'][None, :], p['h2c0_be1'][None, :], p['h2c0_W2'],
               p['h2c0_b2'][None, :])

    ap2 = _sc_rowsum_C128(g, edge_I2[0], edge_I2[1])
    gpad = jnp.pad(g, ((0, 0), (0, 144)))
    h2, h2sum = _tc(_ginc_body,
                    (jax.ShapeDtypeStruct((N_C2, 128), f32),
                     jax.ShapeDtypeStruct((1, 128), f32)),
                    gpad, jnp.pad(ap2, ((0, 0), (0, 0), (0, 144))),
                    jnp.pad(p['h2c1_W1'], ((0, 132), (0, 0))),
                    p['h2c1_b1'][None, :], p['h2c1_g1'][None, :],
                    p['h2c1_be1'][None, :], p['h2c1_W2'],
                    p['h2c1_b2'][None, :])

    h1m = jnp.concatenate(hsums, axis=0).reshape(1, 256) * (1.0 / N_A)
    h2m = h2sum * (1.0 / N_C2)
    hh = jnp.concatenate([h1m, h2m], axis=-1)
    o = _tc(_out_mlp_kernel, jax.ShapeDtypeStruct((1, 1), f32),
            hh, p['out_W1'], p['out_b1'][None, :], p['out_W2'],
            p['out_b2'][None, :])
    return o


# speedup vs baseline: 1.2770x; 1.2770x over previous
import jax
import jax.numpy as jnp
import numpy as np
from jax import lax
from jax.experimental import pallas as pl
from jax.experimental.pallas import tpu as pltpu
from jax.experimental.pallas import tpu_sc as plsc

N_A = 10000
N_C2 = 2000
HID = 128
HEADS = 4
E_B = 320000
E_I2 = 32000

NC = 2
NS = 16

_CP = pltpu.CompilerParams(use_tc_tiling_on_sc=False, needs_layout_passes=False)


def _mesh():
    return plsc.VectorSubcoreMesh(core_axis_name="c", subcore_axis_name="s",
                                  num_cores=NC, num_subcores=NS)


def _zero16(ref, n):
    cgrp = ref.shape[1] // 16

    def it(i, _):
        ref[i // cgrp, pl.ds((i % cgrp) * 16, 16)] = jnp.zeros((16,), jnp.float32)
        return 0
    lax.fori_loop(0, n * cgrp, it, 0)


def _make_sc_rowsum(N, C, E, B, name):
    assert C % 16 == 0 and (C * 4) % 64 == 0 and N % 16 == 0
    assert E % (NC * NS * B) == 0 and B <= 128 and B % 8 == 0
    nchunks = N // 16
    iters = (nchunks + NS - 1) // NS
    e_per_tile = E // (NC * NS)
    nb = e_per_tile // B

    def body(table, src, dst, out, acc, zv, sv, dv, rows0, rows1, sm0, sm1):
        c = lax.axis_index("c")
        s = lax.axis_index("s")
        _zero16(zv, 16)

        def zchunk(i, _):
            k = i * NS + s

            @pl.when(k < nchunks)
            def _():
                pltpu.sync_copy(zv, acc.at[pl.ds(k * 16, 16)])
            return 0
        lax.fori_loop(0, iters, zchunk, 0)
        e0 = c * (E // NC) + s * e_per_tile
        pltpu.sync_copy(src.at[pl.ds(e0, e_per_tile)], sv)
        pltpu.sync_copy(dst.at[pl.ds(e0, e_per_tile)], dv)
        plsc.subcore_barrier()

        pltpu.async_copy(table.at[sv.at[pl.ds(0, B)]], rows0, sm0)

        def batch(j, _):
            def step(cur, csem, nxt, nsem):
                pltpu.make_async_copy(table.at[sv.at[pl.ds(0, B)]],
                                      cur, csem).wait()

                @pl.when(j + 1 < nb)
                def _():
                    pltpu.async_copy(
                        table.at[sv.at[pl.ds((j + 1) * B, B)]], nxt, nsem)
                pltpu.sync_copy(cur, acc.at[dv.at[pl.ds(j * B, B)]], add=True)

            @pl.when(j % 2 == 0)
            def _():
                step(rows0, sm0, rows1, sm1)

            @pl.when(j % 2 == 1)
            def _():
                step(rows1, sm1, rows0, sm0)
            return 0
        lax.fori_loop(0, nb, batch, 0)
        plsc.subcore_barrier()

        def wchunk(i, _):
            k = i * NS + s

            @pl.when(k < nchunks)
            def _():
                pltpu.sync_copy(acc.at[pl.ds(k * 16, 16)],
                                out.at[c, pl.ds(k * 16, 16)])
            return 0
        lax.fori_loop(0, iters, wchunk, 0)

    return pl.kernel(
        body,
        out_type=jax.ShapeDtypeStruct((NC, N, C), jnp.float32),
        mesh=_mesh(),
        compiler_params=_CP,
        scratch_types=[
            pltpu.VMEM_SHARED((N, C), jnp.float32),
            pltpu.VMEM((16, C), jnp.float32),
            pltpu.VMEM((e_per_tile,), jnp.int32),
            pltpu.VMEM((e_per_tile,), jnp.int32),
            pltpu.VMEM((B, C), jnp.float32),
            pltpu.VMEM((B, C), jnp.float32),
            pltpu.SemaphoreType.DMA,
            pltpu.SemaphoreType.DMA,
        ],
        name=name,
    )


def _make_sc_gat(E, B, name):
    assert E % (NS * B) == 0 and B % 16 == 0
    e_per_tile = E // NS
    nb = e_per_tile // B
    nchunks = N_A // 16
    iters = (nchunks + NS - 1) // NS

    def body(X, eler, srcc, dstc, U, den,
             accU, accD, zv, sv, dv, elsrc, eldst, exb, rows, smX, smS, smD):
        c = lax.axis_index("c")
        s = lax.axis_index("s")
        iot = lax.iota(jnp.int32, 16)
        _zero16(zv, 16)

        def zex(i, _):
            exb[i, :] = jnp.zeros((16,), jnp.float32)
            return 0
        lax.fori_loop(0, B, zex, 0)

        def zchunk(i, _):
            k = i * NS + s

            @pl.when(k < nchunks)
            def _():
                pltpu.sync_copy(zv, accU.at[pl.ds(k * 16, 16)])
                pltpu.sync_copy(zv.at[:, pl.ds(0, 16)],
                                accD.at[pl.ds(k * 16, 16)])
            return 0
        lax.fori_loop(0, iters, zchunk, 0)
        plsc.subcore_barrier()

        def batch(j, _):
            e0 = c * E + s * e_per_tile + j * B
            pltpu.sync_copy(srcc.at[pl.ds(e0, B)], sv)
            pltpu.sync_copy(dstc.at[pl.ds(e0, B)], dv)
            cpX = pltpu.async_copy(X.at[c].at[sv], rows, smX)
            cpS = pltpu.async_copy(eler.at[c].at[sv], elsrc, smS)
            cpD = pltpu.async_copy(eler.at[c].at[dv], eldst, smD)
            cpX.wait()
            cpS.wait()
            cpD.wait()
            for i5 in range(B // 16):
                ridx = i5 * 16 + iot
                for h in range(4):
                    el_s = plsc.load_gather(
                        elsrc, [ridx, jnp.full((16,), h, jnp.int32)])
                    er_d = plsc.load_gather(
                        eldst, [ridx, jnp.full((16,), h + 4, jnp.int32)])
                    z = el_s + er_d
                    z = jnp.where(z >= 0.0, z, 0.2 * z)
                    plsc.store_scatter(
                        exb, [ridx, jnp.full((16,), h, jnp.int32)], jnp.exp(z))
            pltpu.sync_copy(exb, accD.at[dv], add=True)

            def scale(i, _):
                for h in range(4):
                    sc = plsc.load_gather(exb, [jnp.full((16,), i, jnp.int32),
                                                jnp.full((16,), h, jnp.int32)])
                    for gg in (2 * h, 2 * h + 1):
                        v = rows[i, pl.ds(gg * 16, 16)]
                        rows[i, pl.ds(gg * 16, 16)] = v * sc
                return 0
            lax.fori_loop(0, B, scale, 0)
            pltpu.sync_copy(rows, accU.at[dv], add=True)
            return 0
        lax.fori_loop(0, nb, batch, 0)
        plsc.subcore_barrier()

        def wchunk(i, _):
            k = i * NS + s

            @pl.when(k < nchunks)
            def _():
                pltpu.sync_copy(accU.at[pl.ds(k * 16, 16)],
                                U.at[c, pl.ds(k * 16, 16)])
                pltpu.sync_copy(accD.at[pl.ds(k * 16, 16)],
                                den.at[c, pl.ds(k * 16, 16)])
            return 0
        lax.fori_loop(0, iters, wchunk, 0)

    return pl.kernel(
        body,
        out_type=(jax.ShapeDtypeStruct((NC, N_A, 128), jnp.float32),
                  jax.ShapeDtypeStruct((NC, N_A, 16), jnp.float32)),
        mesh=_mesh(),
        compiler_params=_CP,
        scratch_types=[
            pltpu.VMEM_SHARED((N_A, 128), jnp.float32),
            pltpu.VMEM_SHARED((N_A, 16), jnp.float32),
            pltpu.VMEM((16, 128), jnp.float32),
            pltpu.VMEM((B,), jnp.int32),
            pltpu.VMEM((B,), jnp.int32),
            pltpu.VMEM((B, 16), jnp.float32),
            pltpu.VMEM((B, 16), jnp.float32),
            pltpu.VMEM((B, 16), jnp.float32),
            pltpu.VMEM((B, 128), jnp.float32),
            pltpu.SemaphoreType.DMA,
            pltpu.SemaphoreType.DMA,
            pltpu.SemaphoreType.DMA,
        ],
        name=name,
    )


def _make_sc_segmax(name):
    def body(h, dstg, out, colsv, dvv, acc):
        c = lax.axis_index("c")
        s = lax.axis_index("s")
        iot = lax.iota(jnp.int32, 16)
        msk = iot < 8
        pltpu.sync_copy(h.at[c, :, pl.ds(8 * s, 8)], colsv)
        pltpu.sync_copy(dstg.at[pl.ds(0, N_A)], dvv)

        def z(i, _):
            acc[i, :] = jnp.zeros((16,), jnp.float32)
            return 0
        lax.fori_loop(0, N_C2, z, 0)

        def it(i, _):
            ivec = jnp.full((16,), i, jnp.int32)
            dvec = plsc.load_gather(dvv, [ivec])
            dval = plsc.load_gather(colsv, [ivec, iot], mask=msk)
            av = plsc.load_gather(acc, [dvec, iot], mask=msk)
            plsc.store_scatter(acc, [dvec, iot], jnp.maximum(av, dval), mask=msk)
            return 0
        lax.fori_loop(0, N_A, it, 0)
        pltpu.sync_copy(acc.at[:, pl.ds(0, 8)], out.at[c, :, pl.ds(8 * s, 8)])

    return pl.kernel(
        body,
        out_type=jax.ShapeDtypeStruct((NC, N_C2, 128), jnp.float32),
        mesh=_mesh(),
        compiler_params=_CP,
        scratch_types=[
            pltpu.VMEM((N_A, 8), jnp.float32),
            pltpu.VMEM((N_A,), jnp.int32),
            pltpu.VMEM((N_C2, 16), jnp.float32),
        ],
        name=name,
    )


_sc_rowsum_A = _make_sc_rowsum(N_A, 128, E_B, 80, "sc_rowsum_a")
_sc_rowsum_C272 = _make_sc_rowsum(N_C2, 272, E_I2, 40, "sc_rowsum_c272")
_sc_rowsum_C128 = _make_sc_rowsum(N_C2, 128, E_I2, 40, "sc_rowsum_c128")
_sc_gat = _make_sc_gat(E_B, 80, "sc_gat")
_sc_segmax = _make_sc_segmax("sc_segmax")


def _mm(a, b):
    return jnp.matmul(a, b, precision='highest')


def _xe_body(f_ref, W_ref, A_ref, x_o, e_o):
    Xt = _mm(f_ref[:], W_ref[:])
    x_o[:] = Xt
    e_o[:] = _mm(Xt, A_ref[:])


def _hgat_body(u_ref, den_ref, b_ref, M_ref, o_ref):
    rec = 1.0 / jnp.maximum(den_ref[:], 1e-9)
    recx = _mm(rec, M_ref[:])
    o_ref[:] = jnp.maximum(u_ref[:] * recx + b_ref[:], 0.0)


def _gin_body(xg_ref, xf_ref, gp_ref, fp_ref, W1a, W1b, b1, g1, be1, W2, b2,
              h_o, s_o):
    xg = xg_ref[:] + gp_ref[0] + gp_ref[1]
    xf = xf_ref[:] + fp_ref[0] + fp_ref[1]
    h = _mm(xg, W1a[:]) + _mm(xf, W1b[:]) + b1[:]
    mu = jnp.mean(h, axis=0)
    d = h - mu
    var = jnp.mean(d * d, axis=0)
    a = jnp.maximum(d / jnp.sqrt(var + 1e-5) * g1[:] + be1[:], 0.0)
    o = jnp.maximum(_mm(a, W2[:]) + b2[:], 0.0)
    h_o[:] = o
    s_o[:] = jnp.sum(o, axis=0, keepdims=True)


def _ginc_body(x_ref, ap_ref, W1, b1, g1, be1, W2, b2, h_o, s_o):
    x = (x_ref[:] + ap_ref[0] + ap_ref[1])[:, :260]
    h = _mm(x, W1[:]) + b1[:]
    mu = jnp.mean(h, axis=0)
    d = h - mu
    var = jnp.mean(d * d, axis=0)
    a = jnp.maximum(d / jnp.sqrt(var + 1e-5) * g1[:] + be1[:], 0.0)
    o = jnp.maximum(_mm(a, W2[:]) + b2[:], 0.0)
    h_o[:] = o
    s_o[:] = jnp.sum(o, axis=0, keepdims=True)


def _out_mlp_kernel(hh_ref, w1_ref, b1_ref, w2_ref, b2_ref, o_ref):
    h = jnp.maximum(_mm(hh_ref[:], w1_ref[:]) + b1_ref[:], 0.0)
    o_ref[:] = _mm(h, w2_ref[:]) + b2_ref[:]


def _tc(body, outs, *args):
    return pl.pallas_call(body, out_shape=outs)(*args)


def kernel(feats_A, pca_C2, rni, params, edge_B1, edge_B2, dst_G1, edge_I2):
    p = params
    f32 = jnp.float32

    feats = jnp.concatenate([feats_A, rni], axis=-1)
    srccat = jnp.concatenate([edge_B1[0], edge_B2[0]])
    dstcat = jnp.concatenate([edge_B1[1], edge_B2[1]])

    Mnp = np.zeros((16, 128), np.float32)
    for h in range(4):
        Mnp[h, 32 * h:32 * h + 32] = 1.0
    M = jnp.asarray(Mnp)

    Xs, elers = [], []
    for t in range(2):
        al, ar = p['gat%d_al' % t], p['gat%d_ar' % t]
        A = jnp.zeros((128, 128), f32)
        for h in range(4):
            A = lax.dynamic_update_slice(A, al[h][:, None], (32 * h, h))
            A = lax.dynamic_update_slice(A, ar[h][:, None], (32 * h, h + 4))
        Xt, et = _tc(_xe_body,
                     (jax.ShapeDtypeStruct((N_A, 128), f32),
                      jax.ShapeDtypeStruct((N_A, 128), f32)),
                     feats, p['gat%d_W' % t], A)
        Xs.append(Xt)
        elers.append(et[:, :16])
    X = jnp.stack(Xs)
    eler = jnp.stack(elers)

    U, den = _sc_gat(X, eler, srccat, dstcat)

    hs, hsums = [], []
    for t, ei in enumerate([edge_B1, edge_B2]):
        src, dst = ei[0], ei[1]
        hgat_t = _tc(_hgat_body, jax.ShapeDtypeStruct((N_A, 128), f32),
                     U[t], den[t], p['gat%d_b' % t][None, :], M)
        gp = _sc_rowsum_A(hgat_t, src, dst)
        fp = _sc_rowsum_A(feats, src, dst)
        W1 = p['gin%d_W1' % t]
        h_t, hsum_t = _tc(
            _gin_body,
            (jax.ShapeDtypeStruct((N_A, 128), f32),
             jax.ShapeDtypeStruct((1, 128), f32)),
            hgat_t, feats, gp, fp, W1[:128], W1[128:],
            p['gin%d_b1' % t][None, :], p['gin%d_g1' % t][None, :],
            p['gin%d_be1' % t][None, :], p['gin%d_W2' % t],
            p['gin%d_b2' % t][None, :])
        hs.append(h_t)
        hsums.append(hsum_t)

    hstack = jnp.stack(hs)
    hC = _sc_segmax(hstack, dst_G1)

    h272 = jnp.concatenate(
        [hC[0], hC[1], pca_C2[..., :4], jnp.zeros((N_C2, 12), f32)], axis=1)
    ap = _sc_rowsum_C272(h272, edge_I2[0], edge_I2[1])
    g, _ = _tc(_ginc_body,
               (jax.ShapeDtypeStruct((N_C2, 128), f32),
                jax.ShapeDtypeStruct((1, 128), f32)),
               h272, ap, p['h2c0_W1'], p['h2c0_b1'][None, :],
               p['h2c0_g1'][None, :], p['h2c0_be1'][None, :], p['h2c0_W2'],
               p['h2c0_b2'][None, :])

    ap2 = _sc_rowsum_C128(g, edge_I2[0], edge_I2[1])
    gpad = jnp.pad(g, ((0, 0), (0, 144)))
    h2, h2sum = _tc(_ginc_body,
                    (jax.ShapeDtypeStruct((N_C2, 128), f32),
                     jax.ShapeDtypeStruct((1, 128), f32)),
                    gpad, jnp.pad(ap2, ((0, 0), (0, 0), (0, 144))),
                    jnp.pad(p['h2c1_W1'], ((0, 132), (0, 0))),
                    p['h2c1_b1'][None, :], p['h2c1_g1'][None, :],
                    p['h2c1_be1'][None, :], p['h2c1_W2'],
                    p['h2c1_b2'][None, :])

    h1m = jnp.concatenate(hsums, axis=0).reshape(1, 256) * (1.0 / N_A)
    h2m = h2sum * (1.0 / N_C2)
    hh = jnp.concatenate([h1m, h2m], axis=-1)
    o = _tc(_out_mlp_kernel, jax.ShapeDtypeStruct((1, 1), f32),
            hh, p['out_W1'], p['out_b1'][None, :], p['out_W2'],
            p['out_b2'][None, :])
    return o
